# baseline (speedup 1.0000x reference)
import functools

import jax
import jax.numpy as jnp
from jax import lax
from jax.experimental import pallas as pl
from jax.experimental.pallas import tpu as pltpu

_TB = 32
_NBUF = 8
_NCORES = 2


def _fused_gather_kernel(ids_ref, w0_hbm, w1_hbm, w2_hbm, m_in_ref, m_out_ref,
                         s0_ref, o_ref, buf, sems, *, tb, nsteps, nbuf):
    core = pl.program_id(0)
    step = pl.program_id(1)
    base0 = core * (nsteps * tb)

    def issue(blk, slot):
        base = base0 + blk * tb
        for t in range(tb):
            row = ids_ref[base + t]
            pltpu.make_async_copy(
                w0_hbm.at[pl.ds(row, 1)], buf.at[slot, pl.ds(t, 1)],
                sems.at[slot]).start()
            pltpu.make_async_copy(
                w1_hbm.at[pl.ds(row, 1)], buf.at[slot, pl.ds(tb + t, 1)],
                sems.at[slot]).start()
            pltpu.make_async_copy(
                w2_hbm.at[pl.ds(row, 1)], buf.at[slot, pl.ds(2 * tb + t, 1)],
                sems.at[slot]).start()

    @pl.when(step == 0)
    def _():
        for k in range(min(nbuf - 1, nsteps)):
            issue(k, k)

    slot = lax.rem(step, nbuf)

    pltpu.make_async_copy(
        w0_hbm.at[pl.ds(0, 3 * tb)], buf.at[slot], sems.at[slot]).wait()

    @pl.when(step + nbuf - 1 < nsteps)
    def _():
        issue(step + nbuf - 1, lax.rem(step + nbuf - 1, nbuf))

    s0 = s0_ref[0]
    base = base0 + step * tb
    col = jnp.concatenate(
        [m_out_ref[ids_ref[base + t]] for t in range(tb)], axis=0)

    b0 = buf[slot, 0:tb]
    b1 = buf[slot, tb:2 * tb]
    b2 = buf[slot, 2 * tb:3 * tb]
    o_ref[...] = b0 * s0 + b1 * m_in_ref[...] + b2 * col


def kernel(input_ids, w0, w1, w2, scalar_mask, vec_in_mask, vec_out_mask):
    B, S = input_ids.shape
    V, H = w0.shape
    dtype = w0.dtype
    T = B * S

    ids = input_ids.reshape(T).astype(jnp.int32)
    chunk = _NCORES * _TB
    t_pad = pl.cdiv(T, chunk) * chunk
    if t_pad != T:
        ids = jnp.pad(ids, (0, t_pad - T))
    nsteps = t_pad // (_NCORES * _TB)

    m_in = jnp.asarray(vec_in_mask, dtype).reshape(1, H)
    m_out = jnp.asarray(vec_out_mask, dtype).reshape(V, 1, 1)
    s0 = jnp.asarray(scalar_mask, dtype).reshape(1)

    grid_spec = pltpu.PrefetchScalarGridSpec(
        num_scalar_prefetch=1,
        grid=(_NCORES, nsteps),
        in_specs=[
            pl.BlockSpec(memory_space=pl.ANY),
            pl.BlockSpec(memory_space=pl.ANY),
            pl.BlockSpec(memory_space=pl.ANY),
            pl.BlockSpec((1, H), lambda c, s, ids: (0, 0)),
            pl.BlockSpec((V, 1, 1), lambda c, s, ids: (0, 0, 0)),
            pl.BlockSpec(memory_space=pltpu.MemorySpace.SMEM),
        ],
        out_specs=pl.BlockSpec((_TB, H),
                               lambda c, s, ids: (c * nsteps + s, 0)),
        scratch_shapes=[
            pltpu.VMEM((_NBUF, 3 * _TB, H), dtype),
            pltpu.SemaphoreType.DMA((_NBUF,)),
        ],
    )
    out = pl.pallas_call(
        functools.partial(_fused_gather_kernel, tb=_TB, nsteps=nsteps,
                          nbuf=_NBUF),
        out_shape=jax.ShapeDtypeStruct((t_pad, H), dtype),
        grid_spec=grid_spec,
        compiler_params=pltpu.CompilerParams(
            dimension_semantics=("parallel", "arbitrary")),
        name="fused_merge_gather",
    )(ids, w0, w1, w2, m_in, m_out, s0)
    return out[:T].reshape(B, S, H)


# speedup vs baseline: 1.0079x; 1.0079x over previous
import functools

import jax
import jax.numpy as jnp
from jax import lax
from jax.experimental import pallas as pl
from jax.experimental.pallas import tpu as pltpu

_TB = 32
_NBUF = 4
_NCORES = 1


def _fused_gather_kernel(ids_ref, w0_hbm, w1_hbm, w2_hbm, m_in_ref, m_out_ref,
                         s0_ref, o_ref, buf, sems, *, tb, nsteps, nbuf):
    core = pl.program_id(0)
    step = pl.program_id(1)
    base0 = core * (nsteps * tb)

    def issue(blk, slot):
        base = base0 + blk * tb
        for t in range(tb):
            row = ids_ref[base + t]
            pltpu.make_async_copy(
                w0_hbm.at[pl.ds(row, 1)], buf.at[slot, pl.ds(t, 1)],
                sems.at[slot]).start()
            pltpu.make_async_copy(
                w1_hbm.at[pl.ds(row, 1)], buf.at[slot, pl.ds(tb + t, 1)],
                sems.at[slot]).start()
            pltpu.make_async_copy(
                w2_hbm.at[pl.ds(row, 1)], buf.at[slot, pl.ds(2 * tb + t, 1)],
                sems.at[slot]).start()

    @pl.when(step == 0)
    def _():
        for k in range(min(nbuf - 1, nsteps)):
            issue(k, k)

    slot = lax.rem(step, nbuf)

    pltpu.make_async_copy(
        w0_hbm.at[pl.ds(0, 3 * tb)], buf.at[slot], sems.at[slot]).wait()

    @pl.when(step + nbuf - 1 < nsteps)
    def _():
        issue(step + nbuf - 1, lax.rem(step + nbuf - 1, nbuf))

    s0 = s0_ref[0]
    base = base0 + step * tb
    col = jnp.concatenate(
        [m_out_ref[ids_ref[base + t]] for t in range(tb)], axis=0)

    b0 = buf[slot, 0:tb]
    b1 = buf[slot, tb:2 * tb]
    b2 = buf[slot, 2 * tb:3 * tb]
    o_ref[...] = b0 * s0 + b1 * m_in_ref[...] + b2 * col


def kernel(input_ids, w0, w1, w2, scalar_mask, vec_in_mask, vec_out_mask):
    B, S = input_ids.shape
    V, H = w0.shape
    dtype = w0.dtype
    T = B * S

    ids = input_ids.reshape(T).astype(jnp.int32)
    chunk = _NCORES * _TB
    t_pad = pl.cdiv(T, chunk) * chunk
    if t_pad != T:
        ids = jnp.pad(ids, (0, t_pad - T))
    nsteps = t_pad // (_NCORES * _TB)

    m_in = jnp.asarray(vec_in_mask, dtype).reshape(1, H)
    m_out = jnp.asarray(vec_out_mask, dtype).reshape(V, 1, 1)
    s0 = jnp.asarray(scalar_mask, dtype).reshape(1)

    grid_spec = pltpu.PrefetchScalarGridSpec(
        num_scalar_prefetch=1,
        grid=(_NCORES, nsteps),
        in_specs=[
            pl.BlockSpec(memory_space=pl.ANY),
            pl.BlockSpec(memory_space=pl.ANY),
            pl.BlockSpec(memory_space=pl.ANY),
            pl.BlockSpec((1, H), lambda c, s, ids: (0, 0)),
            pl.BlockSpec((V, 1, 1), lambda c, s, ids: (0, 0, 0)),
            pl.BlockSpec(memory_space=pltpu.MemorySpace.SMEM),
        ],
        out_specs=pl.BlockSpec((_TB, H),
                               lambda c, s, ids: (c * nsteps + s, 0)),
        scratch_shapes=[
            pltpu.VMEM((_NBUF, 3 * _TB, H), dtype),
            pltpu.SemaphoreType.DMA((_NBUF,)),
        ],
    )
    out = pl.pallas_call(
        functools.partial(_fused_gather_kernel, tb=_TB, nsteps=nsteps,
                          nbuf=_NBUF),
        out_shape=jax.ShapeDtypeStruct((t_pad, H), dtype),
        grid_spec=grid_spec,
        compiler_params=pltpu.CompilerParams(
            dimension_semantics=("parallel", "arbitrary")),
        name="fused_merge_gather",
    )(ids, w0, w1, w2, m_in, m_out, s0)
    return out[:T].reshape(B, S, H)
